# baseline (speedup 1.0000x reference)
import functools

import jax
import jax.numpy as jnp
from jax import lax
from jax.experimental import pallas as pl
from jax.experimental.pallas import tpu as pltpu
from jax.experimental.pallas import tpu_sc as plsc

EMB_DIM = 64
BATCH = 4096
HIST = 200
TOTAL = BATCH * HIST

_INFO = plsc.get_sparse_core_info()
NC = _INFO.num_cores
NS = _INFO.num_subcores
NW = NC * NS
PER_W = TOTAL // NW
CHUNK = 128
NCH = PER_W // CHUNK


def _gather_body(idx_hbm, table_hbm, out_hbm, idx_v, rows_v, sem):
    c = lax.axis_index("c")
    s = lax.axis_index("s")
    wid = s * NC + c
    base_chunk = wid * NCH

    pltpu.sync_copy(idx_hbm.at[pl.ds(base_chunk, NCH)], idx_v)

    def step(j, carry):
        pltpu.async_copy(table_hbm.at[idx_v.at[j]], rows_v, sem).wait()
        row0 = (base_chunk + j) * CHUNK
        pltpu.sync_copy(rows_v, out_hbm.at[pl.ds(row0, CHUNK)])
        return carry

    lax.fori_loop(0, NCH, step, 0)


@functools.partial(
    pl.kernel,
    out_type=jax.ShapeDtypeStruct((TOTAL, EMB_DIM), jnp.float32),
    mesh=plsc.VectorSubcoreMesh(core_axis_name="c", subcore_axis_name="s"),
    scratch_types=[
        pltpu.VMEM((NCH, CHUNK), jnp.int32),
        pltpu.VMEM((CHUNK, EMB_DIM), jnp.float32),
        pltpu.SemaphoreType.DMA,
    ],
    compiler_params=pltpu.CompilerParams(use_tc_tiling_on_sc=False),
)
def _gather_kernel(idx_hbm, table_hbm, out_hbm, idx_v, rows_v, sem):
    _gather_body(idx_hbm, table_hbm, out_hbm, idx_v, rows_v, sem)


def kernel(indices, table):
    idx2d = indices.reshape(TOTAL // CHUNK, CHUNK)
    out = _gather_kernel(idx2d, table)
    return out.reshape(BATCH, HIST, EMB_DIM)


# speedup vs baseline: 1.1152x; 1.1152x over previous
import functools

import jax
import jax.numpy as jnp
from jax import lax
from jax.experimental import pallas as pl
from jax.experimental.pallas import tpu as pltpu
from jax.experimental.pallas import tpu_sc as plsc

EMB_DIM = 64
BATCH = 4096
HIST = 200
TOTAL = BATCH * HIST

_INFO = plsc.get_sparse_core_info()
NC = _INFO.num_cores
NS = _INFO.num_subcores
NW = NC * NS
PER_W = TOTAL // NW
CHUNK = 128
NCH = PER_W // CHUNK
K = 2
NB = 4
G = NCH // K
R = G // NB


def _gather_body(idx_hbm, table_hbm, out_hbm, idx_v, rows_v, gsem, osem):
    c = lax.axis_index("c")
    s = lax.axis_index("s")
    wid = s * NC + c
    base_chunk = wid * NCH

    pltpu.sync_copy(idx_hbm.at[pl.ds(base_chunk, NCH)], idx_v)

    def fire_gathers(g, b):
        for k in range(K):
            pltpu.async_copy(
                table_hbm.at[idx_v.at[g * K + k]],
                rows_v.at[b, pl.ds(k * CHUNK, CHUNK)],
                gsem,
            )

    def wait_gathers(b):
        for k in range(K):
            pltpu.make_async_copy(
                table_hbm.at[idx_v.at[0]],
                rows_v.at[b, pl.ds(k * CHUNK, CHUNK)],
                gsem,
            ).wait()

    def out_slice(g):
        return out_hbm.at[pl.ds((base_chunk + g * K) * CHUNK, K * CHUNK)]

    for b in range(NB):
        fire_gathers(b, b)

    def outer(r, carry):
        for b in range(NB):
            g = r * NB + b
            wait_gathers(b)
            pltpu.async_copy(rows_v.at[b], out_slice(g), osem)
            pltpu.make_async_copy(rows_v.at[b], out_slice(g), osem).wait()

            @pl.when(r < R - 1)
            def _():
                fire_gathers(g + NB, b)

        return carry

    lax.fori_loop(0, R, outer, 0)


@functools.partial(
    pl.kernel,
    out_type=jax.ShapeDtypeStruct((TOTAL, EMB_DIM), jnp.float32),
    mesh=plsc.VectorSubcoreMesh(core_axis_name="c", subcore_axis_name="s"),
    scratch_types=[
        pltpu.VMEM((NCH, CHUNK), jnp.int32),
        pltpu.VMEM((NB, K * CHUNK, EMB_DIM), jnp.float32),
        pltpu.SemaphoreType.DMA,
        pltpu.SemaphoreType.DMA,
    ],
    compiler_params=pltpu.CompilerParams(use_tc_tiling_on_sc=False),
)
def _gather_kernel(idx_hbm, table_hbm, out_hbm, idx_v, rows_v, gsem, osem):
    _gather_body(idx_hbm, table_hbm, out_hbm, idx_v, rows_v, gsem, osem)


def kernel(indices, table):
    idx2d = indices.reshape(TOTAL // CHUNK, CHUNK)
    out = _gather_kernel(idx2d, table)
    return out.reshape(BATCH, HIST, EMB_DIM)


# speedup vs baseline: 1.1152x; 1.0001x over previous
import functools

import jax
import jax.numpy as jnp
from jax import lax
from jax.experimental import pallas as pl
from jax.experimental.pallas import tpu as pltpu
from jax.experimental.pallas import tpu_sc as plsc

EMB_DIM = 64
BATCH = 4096
HIST = 200
TOTAL = BATCH * HIST

_INFO = plsc.get_sparse_core_info()
NC = _INFO.num_cores
NS = _INFO.num_subcores
NW = NC * NS
PER_W = TOTAL // NW
CHUNK = 128
NCH = PER_W // CHUNK
K = 4
NROUND = NCH // (2 * K)


def _gather_body(idx_hbm, table_hbm, out_hbm,
                 idx_v, buf_a, buf_b, sem_ag, sem_as, sem_bg, sem_bs):
    c = lax.axis_index("c")
    s = lax.axis_index("s")
    wid = s * NC + c
    base_chunk = wid * NCH

    pltpu.sync_copy(idx_hbm.at[pl.ds(base_chunk, NCH)], idx_v)

    def fire_gathers(buf, sem, group):
        for k in range(K):
            pltpu.async_copy(
                table_hbm.at[idx_v.at[group * K + k]],
                buf.at[k],
                sem,
            )

    def wait_gathers(buf, sem):
        for k in range(K):
            pltpu.make_async_copy(
                table_hbm.at[idx_v.at[0]], buf.at[k], sem,
            ).wait()

    def fire_scatters(buf, sem, group):
        for k in range(K):
            g = group * K + k
            pltpu.async_copy(
                buf.at[k],
                out_hbm.at[pl.ds((base_chunk + g) * CHUNK, CHUNK)],
                sem,
            )

    def wait_scatters(buf, sem):
        for k in range(K):
            pltpu.make_async_copy(
                buf.at[k],
                out_hbm.at[pl.ds(base_chunk * CHUNK, CHUNK)],
                sem,
            ).wait()

    fire_gathers(buf_a, sem_ag, 0)

    def round_body(r, carry):
        fire_gathers(buf_b, sem_bg, 2 * r + 1)
        wait_gathers(buf_a, sem_ag)
        fire_scatters(buf_a, sem_as, 2 * r)
        wait_scatters(buf_a, sem_as)

        @pl.when(r + 1 < NROUND)
        def _():
            fire_gathers(buf_a, sem_ag, 2 * r + 2)

        wait_gathers(buf_b, sem_bg)
        fire_scatters(buf_b, sem_bs, 2 * r + 1)
        wait_scatters(buf_b, sem_bs)
        return carry

    lax.fori_loop(0, NROUND, round_body, 0)


@functools.partial(
    pl.kernel,
    out_type=jax.ShapeDtypeStruct((TOTAL, EMB_DIM), jnp.float32),
    mesh=plsc.VectorSubcoreMesh(core_axis_name="c", subcore_axis_name="s"),
    scratch_types=[
        pltpu.VMEM((NCH, CHUNK), jnp.int32),
        pltpu.VMEM((K, CHUNK, EMB_DIM), jnp.float32),
        pltpu.VMEM((K, CHUNK, EMB_DIM), jnp.float32),
        pltpu.SemaphoreType.DMA,
        pltpu.SemaphoreType.DMA,
        pltpu.SemaphoreType.DMA,
        pltpu.SemaphoreType.DMA,
    ],
    compiler_params=pltpu.CompilerParams(use_tc_tiling_on_sc=False),
)
def _gather_kernel(idx_hbm, table_hbm, out_hbm,
                   idx_v, buf_a, buf_b, sem_ag, sem_as, sem_bg, sem_bs):
    _gather_body(idx_hbm, table_hbm, out_hbm,
                 idx_v, buf_a, buf_b, sem_ag, sem_as, sem_bg, sem_bs)


def kernel(indices, table):
    idx2d = indices.reshape(TOTAL // CHUNK, CHUNK)
    out = _gather_kernel(idx2d, table)
    return out.reshape(BATCH, HIST, EMB_DIM)
